# Initial kernel scaffold; baseline (speedup 1.0000x reference)
import jax
import jax.numpy as jnp
from jax.experimental import pallas as pl


def kernel(x, position_weight, level_weight):
    raise NotImplementedError("write your pallas kernel here")


# speedup vs baseline: 4.3591x; 4.3591x over previous
import functools
import jax
import jax.numpy as jnp
from jax.experimental import pallas as pl
from jax.experimental.pallas import tpu as pltpu

_B, _S, _D, _L = 128, 512, 1024, 256


def _body(x_ref, pos_ref, lvl_ref, out_ref):
    v = x_ref[0]
    q = v * jnp.float32(_L - 1) + jnp.float32(0.5)
    idx = jnp.clip(q.astype(jnp.int32), 0, _L - 1)
    iota = jax.lax.broadcasted_iota(jnp.int32, (_L, 1), 0)
    onehot_t = (iota == idx).astype(jnp.bfloat16)
    val = jax.lax.dot_general(
        onehot_t, lvl_ref[:, :], (((0,), (0,)), ((), ())),
        preferred_element_type=jnp.float32)
    bound = val * pos_ref[:, :]
    s = jnp.sum(bound, axis=0, keepdims=True)
    out_ref[0] = jnp.where(s > 0, 1.0, -1.0).astype(jnp.float32)


@jax.jit
def kernel(x, position_weight, level_weight):
    x3 = x.reshape(_B, 1, _S)
    lvl = level_weight.astype(jnp.bfloat16)
    out = pl.pallas_call(
        _body,
        grid=(_B,),
        in_specs=[
            pl.BlockSpec((1, 1, _S), lambda b: (b, 0, 0)),
            pl.BlockSpec((_S, _D), lambda b: (0, 0)),
            pl.BlockSpec((_L, _D), lambda b: (0, 0)),
        ],
        out_specs=pl.BlockSpec((1, 1, _D), lambda b: (b, 0, 0)),
        out_shape=jax.ShapeDtypeStruct((_B, 1, _D), jnp.float32),
    )(x3, position_weight, lvl)
    return out.reshape(_B, _D)
